# Initial kernel scaffold; baseline (speedup 1.0000x reference)
import jax
import jax.numpy as jnp
from jax.experimental import pallas as pl


def kernel(x):
    raise NotImplementedError("write your pallas kernel here")


# speedup vs baseline: 2.8983x; 2.8983x over previous
import functools

import jax
import jax.numpy as jnp
from jax import lax
from jax.experimental import pallas as pl
from jax.experimental.pallas import tpu as pltpu
from jax.experimental.pallas import tpu_sc as plsc

B, N, C = 4, 4096, 2048
NW = 32
KB = C // (NW // B)
R = 64
NCHUNK = N // R
NVREG = KB // 16


def _body(x_hbm, o_hbm, in0, in1, out0, out1, s_in0, s_in1, s_out0, s_out1):
  wid = lax.axis_index("s") * 2 + lax.axis_index("c")
  b = wid // (NW // B)
  k = wid % (NW // B)
  row0 = b * N
  c0 = k * KB

  def src(g):
    return x_hbm.at[pl.ds(row0 + g * R, R), pl.ds(c0, KB)]

  def dst(g):
    return o_hbm.at[pl.ds(row0 + g * R, R), pl.ds(c0, KB)]

  pltpu.make_async_copy(src(0), in0, s_in0).start()
  pltpu.make_async_copy(src(1), in1, s_in1).start()

  def compute_chunk(inb, outb, carry):
    def row(r, carry):
      new = []
      for j in range(NVREG):
        c = carry[j] + inb[r, pl.ds(16 * j, 16)]
        outb[r, pl.ds(16 * j, 16)] = c
        new.append(c)
      return tuple(new)
    return lax.fori_loop(0, R, row, carry)

  def chunk_pair(h, carry):
    for p, (inb, outb, s_in, s_out) in enumerate(
        ((in0, out0, s_in0, s_out0), (in1, out1, s_in1, s_out1))):
      g = 2 * h + p
      pltpu.make_async_copy(src(g), inb, s_in).wait()

      @pl.when(h > 0)
      def _():
        pltpu.make_async_copy(outb, dst(g), s_out).wait()

      carry = compute_chunk(inb, outb, carry)
      pltpu.make_async_copy(outb, dst(g), s_out).start()

      @pl.when(g + 2 < NCHUNK)
      def _():
        pltpu.make_async_copy(src(g + 2), inb, s_in).start()
    return carry

  zeros = tuple(jnp.zeros((16,), jnp.float32) for _ in range(NVREG))
  lax.fori_loop(0, NCHUNK // 2, chunk_pair, zeros)

  pltpu.make_async_copy(out0, dst(NCHUNK - 2), s_out0).wait()
  pltpu.make_async_copy(out1, dst(NCHUNK - 1), s_out1).wait()


_scan = functools.partial(
    pl.kernel,
    out_type=jax.ShapeDtypeStruct((B * N, C), jnp.float32),
    mesh=plsc.VectorSubcoreMesh(core_axis_name="c", subcore_axis_name="s"),
    scratch_types=[
        pltpu.VMEM((R, KB), jnp.float32),
        pltpu.VMEM((R, KB), jnp.float32),
        pltpu.VMEM((R, KB), jnp.float32),
        pltpu.VMEM((R, KB), jnp.float32),
        pltpu.SemaphoreType.DMA,
        pltpu.SemaphoreType.DMA,
        pltpu.SemaphoreType.DMA,
        pltpu.SemaphoreType.DMA,
    ],
)(_body)


@jax.jit
def kernel(x):
  out = _scan(x.reshape(B * N, C))
  return out.reshape(B, N, C)
